# baseline (speedup 1.0000x reference)
import functools

import jax
import jax.numpy as jnp
from jax import lax
from jax.experimental import pallas as pl
from jax.experimental.pallas import tpu as pltpu
from jax.experimental.pallas import tpu_sc as plsc

NUM_EMBEDDINGS = 1000000
EMBEDDING_DIM = 16
BATCH = 16384

NUM_CORES = 2
NUM_SUBCORES = 16
NUM_WORKERS = NUM_CORES * NUM_SUBCORES
B_PER_W = BATCH // NUM_WORKERS


def _sc_gather_call(x, table):
    mesh = plsc.VectorSubcoreMesh(core_axis_name="c", subcore_axis_name="s")

    @functools.partial(
        pl.kernel,
        out_type=jax.ShapeDtypeStruct((BATCH, EMBEDDING_DIM), jnp.float32),
        mesh=mesh,
        scratch_types=[
            pltpu.VMEM((B_PER_W,), jnp.int32),
            pltpu.VMEM((B_PER_W, EMBEDDING_DIM), jnp.float32),
            pltpu.SemaphoreType.DMA,
        ],
        compiler_params=pltpu.CompilerParams(use_tc_tiling_on_sc=False),
    )
    def sc_gather(x_hbm, table_hbm, out_hbm, idx_v, rows_v, sem):
        wid = lax.axis_index("s") * NUM_CORES + lax.axis_index("c")
        base = wid * B_PER_W
        pltpu.sync_copy(x_hbm.at[pl.ds(base, B_PER_W)], idx_v)
        pltpu.async_copy(table_hbm.at[idx_v], rows_v, sem).wait()
        pltpu.sync_copy(rows_v, out_hbm.at[pl.ds(base, B_PER_W)])

    return sc_gather(x, table)


def kernel(x, table):
    return _sc_gather_call(x, table)


# speedup vs baseline: 4.8580x; 4.8580x over previous
import functools

import jax
import jax.numpy as jnp
from jax import lax
from jax.experimental import pallas as pl
from jax.experimental.pallas import tpu as pltpu
from jax.experimental.pallas import tpu_sc as plsc

NUM_EMBEDDINGS = 1000000
EMBEDDING_DIM = 16
BATCH = 16384

NUM_CORES = 2
NUM_SUBCORES = 16
NUM_WORKERS = NUM_CORES * NUM_SUBCORES
B_PER_W = BATCH // NUM_WORKERS
LANES = 16
BLOCK = 128
GROUPS = B_PER_W // LANES


def _sc_gather_call(x, table_t):
    mesh = plsc.VectorSubcoreMesh(core_axis_name="c", subcore_axis_name="s")

    scratch = [pltpu.VMEM((B_PER_W,), jnp.int32)]
    scratch += [pltpu.VMEM((LANES, BLOCK), jnp.float32) for _ in range(LANES)]
    scratch += [pltpu.VMEM((EMBEDDING_DIM * B_PER_W,), jnp.float32)]
    scratch += [pltpu.SemaphoreType.DMA]

    @functools.partial(
        pl.kernel,
        out_type=jax.ShapeDtypeStruct((EMBEDDING_DIM, BATCH), jnp.float32),
        mesh=mesh,
        scratch_types=scratch,
        compiler_params=pltpu.CompilerParams(needs_layout_passes=False),
    )
    def sc_gather(x_hbm, table_t_hbm, out_t_hbm, idx_v, *rest):
        bufs = rest[:LANES]
        out_flat = rest[LANES]
        sem = rest[LANES + 1]
        wid = lax.axis_index("s") * NUM_CORES + lax.axis_index("c")
        base = wid * B_PER_W
        pltpu.sync_copy(x_hbm.at[pl.ds(base, B_PER_W)], idx_v)
        iota = lax.iota(jnp.int32, LANES)

        @pl.loop(0, GROUPS)
        def _(g):
            i0 = g * LANES
            v = idx_v[pl.ds(i0, LANES)]
            copies = []
            for j in range(LANES):
                xi = v[j]
                start = pl.multiple_of(xi - (xi & (BLOCK - 1)), BLOCK)
                copies.append(
                    pltpu.async_copy(
                        table_t_hbm.at[:, pl.ds(start, BLOCK)], bufs[j], sem
                    )
                )
            for cp in copies:
                cp.wait()
            for j in range(LANES):
                xi = v[j]
                col = jnp.full_like(iota, xi & (BLOCK - 1))
                row = plsc.load_gather(bufs[j], [iota, col])
                plsc.store_scatter(
                    out_flat, [iota * B_PER_W + (i0 + j)], row
                )

        for c in range(EMBEDDING_DIM):
            pltpu.sync_copy(
                out_flat.at[pl.ds(c * B_PER_W, B_PER_W)],
                out_t_hbm.at[c, pl.ds(base, B_PER_W)],
            )

    return sc_gather(x, table_t)


def kernel(x, table):
    out_t = _sc_gather_call(x, table.T)
    return out_t.T


# speedup vs baseline: 5.8199x; 1.1980x over previous
import functools

import jax
import jax.numpy as jnp
from jax import lax
from jax.experimental import pallas as pl
from jax.experimental.pallas import tpu as pltpu
from jax.experimental.pallas import tpu_sc as plsc

NUM_EMBEDDINGS = 1000000
EMBEDDING_DIM = 16
BATCH = 16384

NUM_CORES = 2
NUM_SUBCORES = 16
NUM_WORKERS = NUM_CORES * NUM_SUBCORES
B_PER_W = BATCH // NUM_WORKERS
LANES = 16
BLOCK = 128
GROUPS = B_PER_W // LANES


def _sc_gather_call(x, table_t):
    mesh = plsc.VectorSubcoreMesh(core_axis_name="c", subcore_axis_name="s")

    scratch = [pltpu.VMEM((B_PER_W,), jnp.int32)]
    scratch += [pltpu.VMEM((LANES, BLOCK), jnp.float32)
                for _ in range(2 * LANES)]
    scratch += [pltpu.VMEM((EMBEDDING_DIM * B_PER_W,), jnp.float32)]
    scratch += [pltpu.SemaphoreType.DMA, pltpu.SemaphoreType.DMA]

    @functools.partial(
        pl.kernel,
        out_type=jax.ShapeDtypeStruct((EMBEDDING_DIM, BATCH), jnp.float32),
        mesh=mesh,
        scratch_types=scratch,
        compiler_params=pltpu.CompilerParams(needs_layout_passes=False),
    )
    def sc_gather(x_hbm, table_t_hbm, out_t_hbm, idx_v, *rest):
        bufs = rest[: 2 * LANES]
        out_flat = rest[2 * LANES]
        sems = rest[2 * LANES + 1:]
        wid = lax.axis_index("s") * NUM_CORES + lax.axis_index("c")
        base = wid * B_PER_W
        pltpu.sync_copy(x_hbm.at[pl.ds(base, B_PER_W)], idx_v)
        iota = lax.iota(jnp.int32, LANES)

        def issue(g, parity):
            v = idx_v[pl.ds(g * LANES, LANES)]
            for j in range(LANES):
                xi = v[j]
                start = pl.multiple_of(xi - (xi & (BLOCK - 1)), BLOCK)
                pltpu.async_copy(
                    table_t_hbm.at[:, pl.ds(start, BLOCK)],
                    bufs[parity * LANES + j],
                    sems[parity],
                )

        def consume(g, parity):
            for j in range(LANES):
                pltpu.make_async_copy(
                    table_t_hbm.at[:, pl.ds(0, BLOCK)],
                    bufs[parity * LANES + j],
                    sems[parity],
                ).wait()
            v = idx_v[pl.ds(g * LANES, LANES)]
            i0 = g * LANES
            for j in range(LANES):
                xi = v[j]
                col = jnp.full_like(iota, xi & (BLOCK - 1))
                row = plsc.load_gather(bufs[parity * LANES + j], [iota, col])
                plsc.store_scatter(
                    out_flat, [iota * B_PER_W + (i0 + j)], row
                )

        issue(0, 0)
        issue(1, 1)

        @pl.loop(0, GROUPS // 2)
        def _(h):
            g0 = h * 2
            consume(g0, 0)

            @pl.when(h < GROUPS // 2 - 1)
            def _():
                issue(g0 + 2, 0)

            consume(g0 + 1, 1)

            @pl.when(h < GROUPS // 2 - 1)
            def _():
                issue(g0 + 3, 1)

        for c in range(EMBEDDING_DIM):
            pltpu.sync_copy(
                out_flat.at[pl.ds(c * B_PER_W, B_PER_W)],
                out_t_hbm.at[c, pl.ds(base, B_PER_W)],
            )

    return sc_gather(x, table_t)


def kernel(x, table):
    out_t = _sc_gather_call(x, table.T)
    return out_t.T


# speedup vs baseline: 6.0578x; 1.0409x over previous
import functools

import jax
import jax.numpy as jnp
from jax import lax
from jax.experimental import pallas as pl
from jax.experimental.pallas import tpu as pltpu
from jax.experimental.pallas import tpu_sc as plsc

NUM_EMBEDDINGS = 1000000
EMBEDDING_DIM = 16
BATCH = 16384

NUM_CORES = 2
NUM_SUBCORES = 16
NUM_WORKERS = NUM_CORES * NUM_SUBCORES
B_PER_W = BATCH // NUM_WORKERS
LANES = 16
BLOCK = 128
SET = 8
NSETS = B_PER_W // SET
DEPTH = 4


def _sc_gather_call(x, table_t):
    mesh = plsc.VectorSubcoreMesh(core_axis_name="c", subcore_axis_name="s")

    scratch = [pltpu.VMEM((B_PER_W,), jnp.int32)]
    scratch += [pltpu.VMEM((LANES, BLOCK), jnp.float32)
                for _ in range(DEPTH * SET)]
    scratch += [pltpu.VMEM((EMBEDDING_DIM * B_PER_W,), jnp.float32)]
    scratch += [pltpu.SemaphoreType.DMA for _ in range(DEPTH)]

    @functools.partial(
        pl.kernel,
        out_type=jax.ShapeDtypeStruct((EMBEDDING_DIM, BATCH), jnp.float32),
        mesh=mesh,
        scratch_types=scratch,
        compiler_params=pltpu.CompilerParams(needs_layout_passes=False),
    )
    def sc_gather(x_hbm, table_t_hbm, out_t_hbm, idx_v, *rest):
        bufs = rest[: DEPTH * SET]
        out_flat = rest[DEPTH * SET]
        sems = rest[DEPTH * SET + 1:]
        wid = lax.axis_index("s") * NUM_CORES + lax.axis_index("c")
        base = wid * B_PER_W
        pltpu.sync_copy(x_hbm.at[pl.ds(base, B_PER_W)], idx_v)
        iota = lax.iota(jnp.int32, LANES)

        def load_half(s, half):
            v = idx_v[pl.ds((s // 2) * LANES, LANES)]
            return v, 8 * half

        def issue(s, p, half):
            v, off = load_half(s, half)
            for j in range(SET):
                xi = v[off + j]
                start = pl.multiple_of(xi - (xi & (BLOCK - 1)), BLOCK)
                pltpu.async_copy(
                    table_t_hbm.at[:, pl.ds(start, BLOCK)],
                    bufs[p * SET + j],
                    sems[p],
                )

        def consume(s, p, half):
            for j in range(SET):
                pltpu.make_async_copy(
                    table_t_hbm.at[:, pl.ds(0, BLOCK)],
                    bufs[p * SET + j],
                    sems[p],
                ).wait()
            v, off = load_half(s, half)
            i0 = s * SET
            for j in range(SET):
                xi = v[off + j]
                col = jnp.full_like(iota, xi & (BLOCK - 1))
                row = plsc.load_gather(bufs[p * SET + j], [iota, col])
                plsc.store_scatter(
                    out_flat, [iota * B_PER_W + (i0 + j)], row
                )

        for s0 in range(DEPTH - 1):
            issue(s0, s0 % DEPTH, s0 % 2)

        @pl.loop(0, NSETS // DEPTH)
        def _(h):
            for q in range(DEPTH):
                s = h * DEPTH + q
                consume(s, q, q % 2)

                @pl.when(s < NSETS - (DEPTH - 1))
                def _():
                    issue(s + DEPTH - 1, (q + DEPTH - 1) % DEPTH,
                          (q + DEPTH - 1) % 2)

        for c in range(EMBEDDING_DIM):
            pltpu.sync_copy(
                out_flat.at[pl.ds(c * B_PER_W, B_PER_W)],
                out_t_hbm.at[c, pl.ds(base, B_PER_W)],
            )

    return sc_gather(x, table_t)


def kernel(x, table):
    out_t = _sc_gather_call(x, table.T)
    return out_t.T
